# baseline (speedup 1.0000x reference)
import functools

import jax
import jax.numpy as jnp
from jax import lax
from jax.experimental import pallas as pl
from jax.experimental.pallas import tpu as pltpu
from jax.experimental.pallas import tpu_sc as plsc

VOCAB = 100000
EMB = 128
TEMPLATES = 1000
BATCH = 4096
SEQ = 50

NC = 2
NS = 16
NW = NC * NS
LANES = 16
NCHUNK = 8

SEQ_PAD = 56
B_PER_W = BATCH // NW
ELEMS_PER_DMA = 2
ROWS_PER_DMA = ELEMS_PER_DMA * SEQ_PAD
CHUNKS = B_PER_W // ELEMS_PER_DMA
NBUF = 4

_MESH = plsc.VectorSubcoreMesh(
    core_axis_name="c", subcore_axis_name="s", num_cores=NC, num_subcores=NS)


@functools.partial(
    pl.kernel,
    out_type=jax.ShapeDtypeStruct((BATCH, EMB), jnp.float32),
    mesh=_MESH,
    scratch_types=[
        pltpu.VMEM((B_PER_W * SEQ_PAD,), jnp.int32),
        [pltpu.VMEM((ROWS_PER_DMA, EMB), jnp.float32) for _ in range(NBUF)],
        pltpu.VMEM((B_PER_W, EMB), jnp.float32),
        [pltpu.SemaphoreType.DMA for _ in range(NBUF)],
    ],
)
def _pool_kernel(ids_hbm, table_hbm, out_hbm, idx_v, rows_bufs, stage_v, sems):
    wid = lax.axis_index("s") * NC + lax.axis_index("c")
    base = pl.multiple_of(wid * (B_PER_W * SEQ_PAD), 8)
    pltpu.sync_copy(ids_hbm.at[pl.ds(base, B_PER_W * SEQ_PAD)], idx_v)

    def issue(c, buf):
        off = pl.multiple_of(c * ROWS_PER_DMA, 8)
        pltpu.make_async_copy(
            table_hbm.at[idx_v.at[pl.ds(off, ROWS_PER_DMA)]],
            rows_bufs[buf], sems[buf],
        ).start()

    for b in range(NBUF - 1):
        issue(jnp.int32(b), b)

    def outer(g, carry):
        for b in range(NBUF):
            c = g * NBUF + b
            nxt = c + NBUF - 1

            @pl.when(nxt < CHUNKS)
            def _():
                issue(nxt, (b + NBUF - 1) % NBUF)

            rows_v = rows_bufs[b]
            pltpu.make_async_copy(
                table_hbm.at[idx_v.at[pl.ds(0, ROWS_PER_DMA)]],
                rows_v, sems[b],
            ).wait()
            for e in range(ELEMS_PER_DMA):
                row0 = e * SEQ_PAD
                accs = [rows_v[row0, pl.ds(k * LANES, LANES)]
                        for k in range(NCHUNK)]
                for s in range(1, SEQ):
                    for k in range(NCHUNK):
                        accs[k] = accs[k] + rows_v[row0 + s,
                                                   pl.ds(k * LANES, LANES)]
                out_row = c * ELEMS_PER_DMA + e
                for k in range(NCHUNK):
                    stage_v[out_row, pl.ds(k * LANES, LANES)] = (
                        accs[k] * jnp.float32(1.0 / SEQ))
        return carry

    lax.fori_loop(0, CHUNKS // NBUF, outer, jnp.int32(0))
    pltpu.sync_copy(stage_v, out_hbm.at[pl.ds(wid * B_PER_W, B_PER_W)])


def _linear_body(mean_ref, w_ref, b_ref, out_ref):
    out_ref[...] = (
        lax.dot_general(
            mean_ref[...], w_ref[...],
            dimension_numbers=(((1,), (1,)), ((), ())),
            preferred_element_type=jnp.float32,
        )
        + b_ref[...]
    )


_B_BLK = 512


def _linear(mean_emb, W, b):
    return pl.pallas_call(
        _linear_body,
        grid=(BATCH // _B_BLK,),
        in_specs=[
            pl.BlockSpec((_B_BLK, EMB), lambda i: (i, 0)),
            pl.BlockSpec((TEMPLATES, EMB), lambda i: (0, 0)),
            pl.BlockSpec((1, TEMPLATES), lambda i: (0, 0)),
        ],
        out_specs=pl.BlockSpec((_B_BLK, TEMPLATES), lambda i: (i, 0)),
        out_shape=jax.ShapeDtypeStruct((BATCH, TEMPLATES), jnp.float32),
    )(mean_emb, W, b.reshape(1, TEMPLATES))


def kernel(input_ids, emb_table, W, b):
    ids = jnp.pad(input_ids.astype(jnp.int32), ((0, 0), (0, SEQ_PAD - SEQ)))
    ids_flat = ids.reshape(BATCH * SEQ_PAD)
    mean_emb = _pool_kernel(ids_flat, emb_table)
    return _linear(mean_emb, W, b)


# speedup vs baseline: 5.0583x; 5.0583x over previous
import functools

import jax
import jax.numpy as jnp
from jax import lax
from jax.experimental import pallas as pl
from jax.experimental.pallas import tpu as pltpu
from jax.experimental.pallas import tpu_sc as plsc

VOCAB = 100000
EMB = 128
TEMPLATES = 1000
BATCH = 4096
SEQ = 50

NC = 2
NS = 16
NW = NC * NS
LANES = 16
NCHUNK = 8

B_PER_W = BATCH // NW
ELEMS_PER_CHUNK = 4
ROWS_PER_CHUNK = ELEMS_PER_CHUNK * SEQ
SPLIT = 128
CHUNKS = B_PER_W // ELEMS_PER_CHUNK
NBUF = 2

_MESH = plsc.VectorSubcoreMesh(
    core_axis_name="c", subcore_axis_name="s", num_cores=NC, num_subcores=NS)


@functools.partial(
    pl.kernel,
    out_type=jax.ShapeDtypeStruct((BATCH, EMB), jnp.float32),
    mesh=_MESH,
    scratch_types=[
        pltpu.VMEM((B_PER_W * SEQ,), jnp.int32),
        [pltpu.VMEM((ROWS_PER_CHUNK, EMB), jnp.float32) for _ in range(NBUF)],
        pltpu.VMEM((B_PER_W, EMB), jnp.float32),
        [pltpu.SemaphoreType.DMA for _ in range(NBUF)],
    ],
)
def _pool_kernel(ids_hbm, table_hbm, out_hbm, idx_v, rows_bufs, stage_v, sems):
    wid = lax.axis_index("s") * NC + lax.axis_index("c")
    base = pl.multiple_of(wid * (B_PER_W * SEQ), 8)
    pltpu.sync_copy(ids_hbm.at[pl.ds(base, B_PER_W * SEQ)], idx_v)

    def issue(c, buf):
        off = pl.multiple_of(c * ROWS_PER_CHUNK, 8)
        pltpu.make_async_copy(
            table_hbm.at[idx_v.at[pl.ds(off, SPLIT)]],
            rows_bufs[buf].at[pl.ds(0, SPLIT)], sems[buf],
        ).start()
        off2 = pl.multiple_of(off + SPLIT, 8)
        pltpu.make_async_copy(
            table_hbm.at[idx_v.at[pl.ds(off2, ROWS_PER_CHUNK - SPLIT)]],
            rows_bufs[buf].at[pl.ds(SPLIT, ROWS_PER_CHUNK - SPLIT)],
            sems[buf],
        ).start()

    for b in range(NBUF - 1):
        issue(jnp.int32(b), b)

    def outer(g, carry):
        for b in range(NBUF):
            c = g * NBUF + b
            nxt = c + NBUF - 1

            @pl.when(nxt < CHUNKS)
            def _():
                issue(nxt, (b + NBUF - 1) % NBUF)

            rows_v = rows_bufs[b]
            pltpu.make_async_copy(
                table_hbm.at[idx_v.at[pl.ds(0, ROWS_PER_CHUNK)]],
                rows_v, sems[b],
            ).wait()
            for e in range(ELEMS_PER_CHUNK):
                row0 = e * SEQ
                accs = [rows_v[row0, pl.ds(k * LANES, LANES)]
                        for k in range(NCHUNK)]
                for s in range(1, SEQ):
                    for k in range(NCHUNK):
                        accs[k] = accs[k] + rows_v[row0 + s,
                                                   pl.ds(k * LANES, LANES)]
                out_row = c * ELEMS_PER_CHUNK + e
                for k in range(NCHUNK):
                    stage_v[out_row, pl.ds(k * LANES, LANES)] = (
                        accs[k] * jnp.float32(1.0 / SEQ))
        return carry

    lax.fori_loop(0, CHUNKS // NBUF, outer, jnp.int32(0))
    pltpu.sync_copy(stage_v, out_hbm.at[pl.ds(wid * B_PER_W, B_PER_W)])


def _linear_body(mean_ref, w_ref, b_ref, out_ref):
    out_ref[...] = (
        lax.dot_general(
            mean_ref[...], w_ref[...],
            dimension_numbers=(((1,), (1,)), ((), ())),
            preferred_element_type=jnp.float32,
        )
        + b_ref[...]
    )


_B_BLK = 512


def _linear(mean_emb, W, b):
    return pl.pallas_call(
        _linear_body,
        grid=(BATCH // _B_BLK,),
        in_specs=[
            pl.BlockSpec((_B_BLK, EMB), lambda i: (i, 0)),
            pl.BlockSpec((TEMPLATES, EMB), lambda i: (0, 0)),
            pl.BlockSpec((1, TEMPLATES), lambda i: (0, 0)),
        ],
        out_specs=pl.BlockSpec((_B_BLK, TEMPLATES), lambda i: (i, 0)),
        out_shape=jax.ShapeDtypeStruct((BATCH, TEMPLATES), jnp.float32),
    )(mean_emb, W, b.reshape(1, TEMPLATES))


def kernel(input_ids, emb_table, W, b):
    ids_flat = input_ids.astype(jnp.int32).reshape(BATCH * SEQ)
    mean_emb = _pool_kernel(ids_flat, emb_table)
    return _linear(mean_emb, W, b)


# speedup vs baseline: 10.3700x; 2.0501x over previous
import functools

import jax
import jax.numpy as jnp
from jax import lax
from jax.experimental import pallas as pl
from jax.experimental.pallas import tpu as pltpu
from jax.experimental.pallas import tpu_sc as plsc

VOCAB = 100000
EMB = 128
TEMPLATES = 1000
BATCH = 4096
SEQ = 50

NC = 2
NS = 16
NW = NC * NS
LANES = 16
NCHUNK = 8

B_PER_W = BATCH // NW
ELEMS_PER_CHUNK = 4
ROWS_PER_CHUNK = ELEMS_PER_CHUNK * SEQ
SPLIT = 128
CHUNKS = B_PER_W // ELEMS_PER_CHUNK
NBUF = 4

_MESH = plsc.VectorSubcoreMesh(
    core_axis_name="c", subcore_axis_name="s", num_cores=NC, num_subcores=NS)


@functools.partial(
    pl.kernel,
    out_type=jax.ShapeDtypeStruct((BATCH, EMB), jnp.float32),
    mesh=_MESH,
    scratch_types=[
        pltpu.VMEM((B_PER_W * SEQ,), jnp.int32),
        [pltpu.VMEM((ROWS_PER_CHUNK, EMB), jnp.float32) for _ in range(NBUF)],
        pltpu.VMEM((B_PER_W, EMB), jnp.float32),
        [pltpu.SemaphoreType.DMA for _ in range(NBUF)],
    ],
)
def _pool_kernel(ids_hbm, table_hbm, out_hbm, idx_v, rows_bufs, stage_v, sems):
    wid = lax.axis_index("s") * NC + lax.axis_index("c")
    base = pl.multiple_of(wid * (B_PER_W * SEQ), 8)
    pltpu.sync_copy(ids_hbm.at[pl.ds(base, B_PER_W * SEQ)], idx_v)

    def issue(c, buf):
        off = pl.multiple_of(c * ROWS_PER_CHUNK, 8)
        pltpu.make_async_copy(
            table_hbm.at[idx_v.at[pl.ds(off, SPLIT)]],
            rows_bufs[buf].at[pl.ds(0, SPLIT)], sems[buf],
        ).start()
        off2 = pl.multiple_of(off + SPLIT, 8)
        pltpu.make_async_copy(
            table_hbm.at[idx_v.at[pl.ds(off2, ROWS_PER_CHUNK - SPLIT)]],
            rows_bufs[buf].at[pl.ds(SPLIT, ROWS_PER_CHUNK - SPLIT)],
            sems[buf],
        ).start()

    for b in range(NBUF - 1):
        issue(jnp.int32(b), b)

    def outer(g, carry):
        for b in range(NBUF):
            c = g * NBUF + b
            nxt = c + NBUF - 1

            @pl.when(nxt < CHUNKS)
            def _():
                issue(nxt, (b + NBUF - 1) % NBUF)

            rows_v = rows_bufs[b]
            pltpu.make_async_copy(
                table_hbm.at[idx_v.at[pl.ds(0, ROWS_PER_CHUNK)]],
                rows_v, sems[b],
            ).wait()
            for e in range(ELEMS_PER_CHUNK):
                row0 = e * SEQ

                def seq_body(s, accs, _row0=row0):
                    return tuple(
                        accs[k] + rows_v[_row0 + s, pl.ds(k * LANES, LANES)]
                        for k in range(NCHUNK)
                    )

                accs = lax.fori_loop(
                    0, SEQ, seq_body,
                    tuple(jnp.zeros((LANES,), jnp.float32)
                          for _ in range(NCHUNK)),
                    unroll=10)
                out_row = c * ELEMS_PER_CHUNK + e
                for k in range(NCHUNK):
                    stage_v[out_row, pl.ds(k * LANES, LANES)] = (
                        accs[k] * jnp.float32(1.0 / SEQ))
        return carry

    lax.fori_loop(0, CHUNKS // NBUF, outer, jnp.int32(0))
    pltpu.sync_copy(stage_v, out_hbm.at[pl.ds(wid * B_PER_W, B_PER_W)])


def _linear_body(mean_ref, w_ref, b_ref, out_ref):
    out_ref[...] = (
        lax.dot_general(
            mean_ref[...], w_ref[...],
            dimension_numbers=(((1,), (1,)), ((), ())),
            preferred_element_type=jnp.float32,
        )
        + b_ref[...]
    )


_B_BLK = 512


def _linear(mean_emb, W, b):
    return pl.pallas_call(
        _linear_body,
        grid=(BATCH // _B_BLK,),
        in_specs=[
            pl.BlockSpec((_B_BLK, EMB), lambda i: (i, 0)),
            pl.BlockSpec((TEMPLATES, EMB), lambda i: (0, 0)),
            pl.BlockSpec((1, TEMPLATES), lambda i: (0, 0)),
        ],
        out_specs=pl.BlockSpec((_B_BLK, TEMPLATES), lambda i: (i, 0)),
        out_shape=jax.ShapeDtypeStruct((BATCH, TEMPLATES), jnp.float32),
    )(mean_emb, W, b.reshape(1, TEMPLATES))


def kernel(input_ids, emb_table, W, b):
    ids_flat = input_ids.astype(jnp.int32).reshape(BATCH * SEQ)
    mean_emb = _pool_kernel(ids_flat, emb_table)
    return _linear(mean_emb, W, b)
